# Initial kernel scaffold; baseline (speedup 1.0000x reference)
import jax
import jax.numpy as jnp
from jax.experimental import pallas as pl


def kernel(input_ids, table):
    raise NotImplementedError("write your pallas kernel here")


# speedup vs baseline: 1.1116x; 1.1116x over previous
import functools

import jax
import jax.numpy as jnp
from jax import lax
from jax.experimental import pallas as pl
from jax.experimental.pallas import tpu as pltpu
from jax.experimental.pallas import tpu_sc as plsc

_NUM_CORES = 2
_NUM_SUBCORES = 16
_NUM_WORKERS = _NUM_CORES * _NUM_SUBCORES
_CHUNK = 1024


@functools.partial(jax.jit, static_argnames=("b_per_w", "chunk"))
def _gather_sc(flat_idx, table, *, b_per_w, chunk):
    total, = flat_idx.shape
    _, dim = table.shape
    n_chunks = b_per_w // chunk
    mesh = plsc.VectorSubcoreMesh(core_axis_name="c", subcore_axis_name="s")

    @functools.partial(
        pl.kernel,
        mesh=mesh,
        out_type=jax.ShapeDtypeStruct((total, dim), jnp.float32),
        scratch_types=[
            pltpu.VMEM((b_per_w,), jnp.int32),
            pltpu.VMEM((2, chunk, dim), jnp.float32),
            pltpu.SemaphoreType.DMA,
            pltpu.SemaphoreType.DMA,
            pltpu.SemaphoreType.DMA,
            pltpu.SemaphoreType.DMA,
        ],
        compiler_params=pltpu.CompilerParams(use_tc_tiling_on_sc=False),
    )
    def k(idx_hbm, table_hbm, out_hbm, idx_v, rows_v, sg0, sg1, ss0, ss1):
        wid = lax.axis_index("s") * _NUM_CORES + lax.axis_index("c")
        base = wid * b_per_w
        pltpu.sync_copy(idx_hbm.at[pl.ds(base, b_per_w)], idx_v)

        sem_g = (sg0, sg1)
        sem_s = (ss0, ss1)
        gathers = [None] * n_chunks
        stores = [None] * n_chunks
        for i in range(n_chunks):
            b = i % 2
            if i >= 2:
                stores[i - 2].wait()
            gathers[i] = pltpu.make_async_copy(
                table_hbm.at[idx_v.at[pl.ds(i * chunk, chunk)]],
                rows_v.at[b],
                sem_g[b],
            )
            gathers[i].start()
            if i >= 1:
                gathers[i - 1].wait()
                stores[i - 1] = pltpu.make_async_copy(
                    rows_v.at[1 - b],
                    out_hbm.at[pl.ds(base + (i - 1) * chunk, chunk)],
                    sem_s[1 - b],
                )
                stores[i - 1].start()
        last = n_chunks - 1
        gathers[last].wait()
        stores[last] = pltpu.make_async_copy(
            rows_v.at[last % 2],
            out_hbm.at[pl.ds(base + last * chunk, chunk)],
            sem_s[last % 2],
        )
        stores[last].start()
        if n_chunks >= 2:
            stores[last - 1].wait()
        stores[last].wait()

    return k(flat_idx, table)


def kernel(input_ids, table):
    batch, hist = input_ids.shape
    _, dim = table.shape
    flat = input_ids.reshape(-1).astype(jnp.int32)
    total = flat.shape[0]
    b_per_w = total // _NUM_WORKERS
    out = _gather_sc(flat, table, b_per_w=b_per_w, chunk=_CHUNK)
    return out.reshape(batch, hist, dim)
